# baseline (speedup 1.0000x reference)
import jax
import jax.numpy as jnp
from jax import lax
from jax.experimental import pallas as pl
from jax.experimental.pallas import tpu as pltpu
from jax.experimental.pallas import tpu_sc as plsc

N_NODES = 10000
N_EDGES = 160000
D = 32
DD = D * D
Q = 4
QD = Q * D
QDD = Q * DD
N_EQ = N_EDGES // Q
N_NQ = N_NODES // Q

_NC = 2
_NS = 16
_NW = _NC * _NS
_CHUNK = 128
_NCHUNKS = N_EDGES // _CHUNK
_SC_MESH = dict(core_axis_name="c", subcore_axis_name="s",
                num_cores=_NC, num_subcores=_NS)


def _gather_body(ns_hbm, src_hbm, out_hbm, idx_v, rows_v, sem):
    wid = lax.axis_index("c") * _NS + lax.axis_index("s")

    def step(t, _):
        chunk = wid + t * _NW

        @pl.when(chunk < _NCHUNKS)
        def _():
            pltpu.sync_copy(src_hbm.at[chunk], idx_v)
            pltpu.async_copy(ns_hbm.at[idx_v], rows_v, sem).wait()
            pltpu.sync_copy(rows_v, out_hbm.at[pl.ds(chunk * _CHUNK, _CHUNK)])
        return _

    lax.fori_loop(0, (_NCHUNKS + _NW - 1) // _NW, step, None)


def _sc_gather(node_states, src2):
    mesh = plsc.VectorSubcoreMesh(**_SC_MESH)
    return pl.kernel(
        _gather_body,
        out_type=jax.ShapeDtypeStruct((N_EDGES, D), jnp.float32),
        mesh=mesh,
        scratch_types=[
            pltpu.VMEM((_CHUNK,), jnp.int32),
            pltpu.VMEM((_CHUNK, D), jnp.float32),
            pltpu.SemaphoreType.DMA,
        ],
        compiler_params=pltpu.CompilerParams(use_tc_tiling_on_sc=False),
    )(node_states, src2)


_ROWS_PER_SUB = N_NODES // _NS


def _scatter_body(msg_hbm, dst_hbm, zs_hbm, ones_hbm, sums_hbm, cnts_hbm,
                  acc_sh, cnt_sh, idx_v, msg_v, ones_v, row_v):
    cid = lax.axis_index("c")
    sid = lax.axis_index("s")
    wid = cid * _NS + sid
    rows0 = sid * _ROWS_PER_SUB

    pltpu.sync_copy(zs_hbm.at[pl.ds(rows0, _ROWS_PER_SUB)], row_v)
    pltpu.sync_copy(row_v, acc_sh.at[pl.ds(rows0, _ROWS_PER_SUB)])
    pltpu.sync_copy(row_v, cnt_sh.at[pl.ds(rows0, _ROWS_PER_SUB)])
    pltpu.sync_copy(ones_hbm, ones_v)
    plsc.subcore_barrier()

    def step(t, _):
        chunk = wid + t * _NW

        @pl.when(chunk < _NCHUNKS)
        def _():
            pltpu.sync_copy(dst_hbm.at[chunk], idx_v)
            pltpu.sync_copy(msg_hbm.at[pl.ds(chunk * _CHUNK, _CHUNK)], msg_v)
            pltpu.sync_copy(msg_v, acc_sh.at[idx_v], add=True)
            pltpu.sync_copy(ones_v, cnt_sh.at[idx_v], add=True)
        return _

    lax.fori_loop(0, (_NCHUNKS + _NW - 1) // _NW, step, None)
    plsc.subcore_barrier()

    pltpu.sync_copy(acc_sh.at[pl.ds(rows0, _ROWS_PER_SUB)], row_v)
    pltpu.sync_copy(row_v, sums_hbm.at[cid, pl.ds(rows0, _ROWS_PER_SUB)])
    pltpu.sync_copy(cnt_sh.at[pl.ds(rows0, _ROWS_PER_SUB)], row_v)
    pltpu.sync_copy(row_v, cnts_hbm.at[cid, pl.ds(rows0, _ROWS_PER_SUB)])


def _sc_scatter(msg, dst2):
    mesh = plsc.VectorSubcoreMesh(**_SC_MESH)
    zs = jnp.zeros((N_NODES, D), jnp.float32)
    ones = jnp.ones((_CHUNK, D), jnp.float32)
    return pl.kernel(
        _scatter_body,
        out_type=(
            jax.ShapeDtypeStruct((_NC, N_NODES, D), jnp.float32),
            jax.ShapeDtypeStruct((_NC, N_NODES, D), jnp.float32),
        ),
        mesh=mesh,
        scratch_types=[
            pltpu.VMEM_SHARED((N_NODES, D), jnp.float32),
            pltpu.VMEM_SHARED((N_NODES, D), jnp.float32),
            pltpu.VMEM((_CHUNK,), jnp.int32),
            pltpu.VMEM((_CHUNK, D), jnp.float32),
            pltpu.VMEM((_CHUNK, D), jnp.float32),
            pltpu.VMEM((_ROWS_PER_SUB, D), jnp.float32),
        ],
        compiler_params=pltpu.CompilerParams(use_tc_tiling_on_sc=False),
    )(msg, dst2, zs, ones)


_MV_BLK = 400


def _matvec_body(x_ref, a_ref, r_ref, out_ref):
    xb = x_ref[...].astype(jnp.bfloat16)
    xrep = jnp.dot(xb, r_ref[...], preferred_element_type=jnp.float32)
    prod = xrep * a_ref[...]
    for g in range(Q):
        t = prod[:, g * DD:g * DD + 128]
        for m in range(1, 8):
            t = t + prod[:, g * DD + m * 128:g * DD + (m + 1) * 128]
        out_ref[:, g * D:(g + 1) * D] = (
            t[:, 0:32] + t[:, 32:64] + t[:, 64:96] + t[:, 96:128])


def _matvec(x4, a4, r4):
    grid = N_EQ // _MV_BLK
    return pl.pallas_call(
        _matvec_body,
        grid=(grid,),
        in_specs=[
            pl.BlockSpec((_MV_BLK, QD), lambda i: (i, 0)),
            pl.BlockSpec((_MV_BLK, QDD), lambda i: (i, 0)),
            pl.BlockSpec((QD, QDD), lambda i: (0, 0)),
        ],
        out_specs=pl.BlockSpec((_MV_BLK, QD), lambda i: (i, 0)),
        out_shape=jax.ShapeDtypeStruct((N_EQ, QD), jnp.float32),
        compiler_params=pltpu.CompilerParams(
            dimension_semantics=("arbitrary",),
        ),
    )(x4, a4, r4)


def _make_r4() -> jax.Array:
    el = jnp.arange(QD)[:, None]
    c = jnp.arange(QDD)[None, :]
    return (el == D * (c // DD) + (c % DD) // D).astype(jnp.bfloat16)


_FIN_BLK = 2500


def _finalize_body(s_ref, c_ref, b_ref, out_ref):
    s = s_ref[0] + s_ref[1]
    c = c_ref[0] + c_ref[1]
    mean = s / jnp.maximum(c, 1.0)
    out_ref[...] = jnp.maximum(mean + b_ref[...], 0.0)


def _finalize(sums4, cnts4, bias4):
    grid = N_NQ // _FIN_BLK
    return pl.pallas_call(
        _finalize_body,
        grid=(grid,),
        in_specs=[
            pl.BlockSpec((_NC, _FIN_BLK, QD), lambda i: (0, i, 0)),
            pl.BlockSpec((_NC, _FIN_BLK, QD), lambda i: (0, i, 0)),
            pl.BlockSpec((1, QD), lambda i: (0, 0)),
        ],
        out_specs=pl.BlockSpec((_FIN_BLK, QD), lambda i: (i, 0)),
        out_shape=jax.ShapeDtypeStruct((N_NQ, QD), jnp.float32),
    )(sums4, cnts4, bias4)


def kernel(node_states, edge_index, edge, a_in, bias):
    del edge
    src = edge_index[:, 0]
    dst = edge_index[:, 1]

    x_i = _sc_gather(node_states, src.reshape(_NCHUNKS, _CHUNK))

    msg4 = _matvec(x_i.reshape(N_EQ, QD), a_in.reshape(N_EQ, QDD), _make_r4())

    sums2, cnts2 = _sc_scatter(msg4.reshape(N_EDGES, D),
                               dst.reshape(_NCHUNKS, _CHUNK))

    out4 = _finalize(sums2.reshape(_NC, N_NQ, QD),
                     cnts2.reshape(_NC, N_NQ, QD),
                     jnp.tile(bias, Q).reshape(1, QD))
    return out4.reshape(N_NODES, D)


# speedup vs baseline: 2.5475x; 2.5475x over previous
import functools

import jax
import jax.numpy as jnp
from jax import lax
from jax.experimental import pallas as pl
from jax.experimental.pallas import tpu as pltpu
from jax.experimental.pallas import tpu_sc as plsc

N_NODES = 10000
N_EDGES = 160000
D = 32
DD = D * D

_NC = 2
_NS = 16
_NW = _NC * _NS
_CHUNK = 128
_NCHUNKS = N_EDGES // _CHUNK
_SC_MESH = dict(core_axis_name="c", subcore_axis_name="s",
                num_cores=_NC, num_subcores=_NS)


def _gather_body(ns_hbm, src_hbm, out_hbm, idx_v, rows_v, sem):
    wid = lax.axis_index("c") * _NS + lax.axis_index("s")

    def step(t, _):
        chunk = wid + t * _NW

        @pl.when(chunk < _NCHUNKS)
        def _():
            pltpu.sync_copy(src_hbm.at[chunk], idx_v)
            pltpu.async_copy(ns_hbm.at[idx_v], rows_v, sem).wait()
            pltpu.sync_copy(rows_v, out_hbm.at[pl.ds(chunk * _CHUNK, _CHUNK)])
        return _

    lax.fori_loop(0, (_NCHUNKS + _NW - 1) // _NW, step, None)


def _sc_gather(node_states, src2):
    mesh = plsc.VectorSubcoreMesh(**_SC_MESH)
    return pl.kernel(
        _gather_body,
        out_type=jax.ShapeDtypeStruct((N_EDGES, D), jnp.float32),
        mesh=mesh,
        scratch_types=[
            pltpu.VMEM((_CHUNK,), jnp.int32),
            pltpu.VMEM((_CHUNK, D), jnp.float32),
            pltpu.SemaphoreType.DMA,
        ],
        compiler_params=pltpu.CompilerParams(use_tc_tiling_on_sc=False),
    )(node_states, src2)


_ROWS_PER_SUB = N_NODES // _NS


def _scatter_body(msg_hbm, dst_hbm, zs_hbm, ones_hbm, sums_hbm, cnts_hbm,
                  acc_sh, cnt_sh, idx_v, msg_v, ones_v, row_v):
    cid = lax.axis_index("c")
    sid = lax.axis_index("s")
    wid = cid * _NS + sid
    rows0 = sid * _ROWS_PER_SUB

    pltpu.sync_copy(zs_hbm.at[pl.ds(rows0, _ROWS_PER_SUB)], row_v)
    pltpu.sync_copy(row_v, acc_sh.at[pl.ds(rows0, _ROWS_PER_SUB)])
    pltpu.sync_copy(row_v, cnt_sh.at[pl.ds(rows0, _ROWS_PER_SUB)])
    pltpu.sync_copy(ones_hbm, ones_v)
    plsc.subcore_barrier()

    def step(t, _):
        chunk = wid + t * _NW

        @pl.when(chunk < _NCHUNKS)
        def _():
            pltpu.sync_copy(dst_hbm.at[chunk], idx_v)
            pltpu.sync_copy(msg_hbm.at[pl.ds(chunk * _CHUNK, _CHUNK)], msg_v)
            pltpu.sync_copy(msg_v, acc_sh.at[idx_v], add=True)
            pltpu.sync_copy(ones_v, cnt_sh.at[idx_v], add=True)
        return _

    lax.fori_loop(0, (_NCHUNKS + _NW - 1) // _NW, step, None)
    plsc.subcore_barrier()

    pltpu.sync_copy(acc_sh.at[pl.ds(rows0, _ROWS_PER_SUB)], row_v)
    pltpu.sync_copy(row_v, sums_hbm.at[cid, pl.ds(rows0, _ROWS_PER_SUB)])
    pltpu.sync_copy(cnt_sh.at[pl.ds(rows0, _ROWS_PER_SUB)], row_v)
    pltpu.sync_copy(row_v, cnts_hbm.at[cid, pl.ds(rows0, _ROWS_PER_SUB)])


def _sc_scatter(msg, dst2):
    mesh = plsc.VectorSubcoreMesh(**_SC_MESH)
    zs = jnp.zeros((N_NODES, D), jnp.float32)
    ones = jnp.ones((_CHUNK, D), jnp.float32)
    return pl.kernel(
        _scatter_body,
        out_type=(
            jax.ShapeDtypeStruct((_NC, N_NODES, D), jnp.float32),
            jax.ShapeDtypeStruct((_NC, N_NODES, D), jnp.float32),
        ),
        mesh=mesh,
        scratch_types=[
            pltpu.VMEM_SHARED((N_NODES, D), jnp.float32),
            pltpu.VMEM_SHARED((N_NODES, D), jnp.float32),
            pltpu.VMEM((_CHUNK,), jnp.int32),
            pltpu.VMEM((_CHUNK, D), jnp.float32),
            pltpu.VMEM((_CHUNK, D), jnp.float32),
            pltpu.VMEM((_ROWS_PER_SUB, D), jnp.float32),
        ],
        compiler_params=pltpu.CompilerParams(use_tc_tiling_on_sc=False),
    )(msg, dst2, zs, ones)


_MV_BLK = 2000


def _matvec_body(x_ref, a_ref, r_ref, out_ref):
    xb = x_ref[...].astype(jnp.bfloat16)
    xrep = jnp.dot(xb, r_ref[...], preferred_element_type=jnp.float32)
    prod = xrep * a_ref[...]
    t = prod[:, 0:128]
    for g in range(1, 8):
        t = t + prod[:, g * 128:(g + 1) * 128]
    out_ref[...] = (t[:, 0:32] + t[:, 32:64] + t[:, 64:96] + t[:, 96:128])


def _matvec(x_i, a2, r_mat):
    grid = N_EDGES // _MV_BLK
    return pl.pallas_call(
        _matvec_body,
        grid=(grid,),
        in_specs=[
            pl.BlockSpec((_MV_BLK, D), lambda i: (i, 0)),
            pl.BlockSpec((_MV_BLK, DD), lambda i: (i, 0)),
            pl.BlockSpec((D, DD), lambda i: (0, 0)),
        ],
        out_specs=pl.BlockSpec((_MV_BLK, D), lambda i: (i, 0)),
        out_shape=jax.ShapeDtypeStruct((N_EDGES, D), jnp.float32),
        compiler_params=pltpu.CompilerParams(
            dimension_semantics=("arbitrary",),
        ),
    )(x_i, a2, r_mat)


def _make_r() -> jax.Array:
    k = jnp.arange(D)[:, None]
    c = jnp.arange(DD)[None, :]
    return (c // D == k).astype(jnp.bfloat16)


_FIN_BLK = 2000


def _finalize_body(s_ref, c_ref, b_ref, out_ref):
    s = s_ref[0] + s_ref[1]
    c = c_ref[0] + c_ref[1]
    mean = s / jnp.maximum(c, 1.0)
    out_ref[...] = jnp.maximum(mean + b_ref[...], 0.0)


def _finalize(sums, counts, bias):
    grid = N_NODES // _FIN_BLK
    return pl.pallas_call(
        _finalize_body,
        grid=(grid,),
        in_specs=[
            pl.BlockSpec((2, _FIN_BLK, D), lambda i: (0, i, 0)),
            pl.BlockSpec((2, _FIN_BLK, D), lambda i: (0, i, 0)),
            pl.BlockSpec((1, D), lambda i: (0, 0)),
        ],
        out_specs=pl.BlockSpec((_FIN_BLK, D), lambda i: (i, 0)),
        out_shape=jax.ShapeDtypeStruct((N_NODES, D), jnp.float32),
    )(sums, counts, bias.reshape(1, D))


def kernel(node_states, edge_index, edge, a_in, bias):
    del edge
    src = edge_index[:, 0]
    dst = edge_index[:, 1]
    a2 = a_in.reshape(N_EDGES, DD)

    x_i = _sc_gather(node_states, src.reshape(_NCHUNKS, _CHUNK))

    msg = _matvec(x_i, a2, _make_r())

    sums2, counts2 = _sc_scatter(msg, dst.reshape(_NCHUNKS, _CHUNK))
    return _finalize(sums2, counts2, bias)


# speedup vs baseline: 2.6309x; 1.0327x over previous
import functools

import jax
import jax.numpy as jnp
from jax import lax
from jax.experimental import pallas as pl
from jax.experimental.pallas import tpu as pltpu
from jax.experimental.pallas import tpu_sc as plsc

N_NODES = 10000
N_EDGES = 160000
D = 32
DD = D * D

_NC = 2
_NS = 16
_NW = _NC * _NS
_CHUNK = 128
_NCHUNKS = N_EDGES // _CHUNK
_SUP = 10
_NSUP = _NCHUNKS // _SUP
_SUPE = _SUP * _CHUNK
_SC_MESH = dict(core_axis_name="c", subcore_axis_name="s",
                num_cores=_NC, num_subcores=_NS)


def _gather_body(ns_hbm, src_hbm, out_hbm, idx_v, rows_v, sem):
    wid = lax.axis_index("c") * _NS + lax.axis_index("s")

    def step(t, _):
        sup = wid + t * _NW

        @pl.when(sup < _NSUP)
        def _():
            pltpu.sync_copy(src_hbm.at[sup], idx_v)
            for j in range(_SUP):
                pltpu.async_copy(
                    ns_hbm.at[idx_v.at[j]],
                    rows_v.at[pl.ds(j * _CHUNK, _CHUNK)], sem)
            for j in range(_SUP):
                pltpu.make_async_copy(
                    ns_hbm.at[idx_v.at[j]],
                    rows_v.at[pl.ds(j * _CHUNK, _CHUNK)], sem).wait()
            pltpu.sync_copy(rows_v, out_hbm.at[pl.ds(sup * _SUPE, _SUPE)])
        return _

    lax.fori_loop(0, (_NSUP + _NW - 1) // _NW, step, None)


def _sc_gather(node_states, src3):
    mesh = plsc.VectorSubcoreMesh(**_SC_MESH)
    return pl.kernel(
        _gather_body,
        out_type=jax.ShapeDtypeStruct((N_EDGES, D), jnp.float32),
        mesh=mesh,
        scratch_types=[
            pltpu.VMEM((_SUP, _CHUNK), jnp.int32),
            pltpu.VMEM((_SUPE, D), jnp.float32),
            pltpu.SemaphoreType.DMA,
        ],
        compiler_params=pltpu.CompilerParams(use_tc_tiling_on_sc=False),
    )(node_states, src3)


_ROWS_PER_SUB = N_NODES // _NS


def _scatter_body(msg_hbm, dst_hbm, zs_hbm, ones_hbm, sums_hbm, cnts_hbm,
                  acc_sh, cnt_sh, idx_v, msg_v, ones_v, row_v, sem, sem2):
    cid = lax.axis_index("c")
    sid = lax.axis_index("s")
    wid = cid * _NS + sid
    rows0 = sid * _ROWS_PER_SUB

    pltpu.sync_copy(zs_hbm.at[pl.ds(rows0, _ROWS_PER_SUB)], row_v)
    pltpu.sync_copy(row_v, acc_sh.at[pl.ds(rows0, _ROWS_PER_SUB)])
    pltpu.sync_copy(row_v, cnt_sh.at[pl.ds(rows0, _ROWS_PER_SUB)])
    pltpu.sync_copy(ones_hbm, ones_v)
    plsc.subcore_barrier()

    def step(t, _):
        sup = wid + t * _NW

        @pl.when(sup < _NSUP)
        def _():
            pltpu.sync_copy(dst_hbm.at[sup], idx_v)
            pltpu.sync_copy(msg_hbm.at[pl.ds(sup * _SUPE, _SUPE)], msg_v)
            for j in range(_SUP):
                pltpu.async_copy(
                    msg_v.at[pl.ds(j * _CHUNK, _CHUNK)],
                    acc_sh.at[idx_v.at[j]], sem, add=True)
                pltpu.async_copy(
                    ones_v, cnt_sh.at[idx_v.at[j]], sem2, add=True)
            for j in range(_SUP):
                pltpu.make_async_copy(
                    msg_v.at[pl.ds(j * _CHUNK, _CHUNK)],
                    acc_sh.at[idx_v.at[j]], sem).wait()
                pltpu.make_async_copy(
                    ones_v, cnt_sh.at[idx_v.at[j]], sem2).wait()
        return _

    lax.fori_loop(0, (_NSUP + _NW - 1) // _NW, step, None)
    plsc.subcore_barrier()

    pltpu.sync_copy(acc_sh.at[pl.ds(rows0, _ROWS_PER_SUB)], row_v)
    pltpu.sync_copy(row_v, sums_hbm.at[cid, pl.ds(rows0, _ROWS_PER_SUB)])
    pltpu.sync_copy(cnt_sh.at[pl.ds(rows0, _ROWS_PER_SUB)], row_v)
    pltpu.sync_copy(row_v, cnts_hbm.at[cid, pl.ds(rows0, _ROWS_PER_SUB)])


def _sc_scatter(msg, dst2):
    mesh = plsc.VectorSubcoreMesh(**_SC_MESH)
    zs = jnp.zeros((N_NODES, D), jnp.float32)
    ones = jnp.ones((_CHUNK, D), jnp.float32)
    return pl.kernel(
        _scatter_body,
        out_type=(
            jax.ShapeDtypeStruct((_NC, N_NODES, D), jnp.float32),
            jax.ShapeDtypeStruct((_NC, N_NODES, D), jnp.float32),
        ),
        mesh=mesh,
        scratch_types=[
            pltpu.VMEM_SHARED((N_NODES, D), jnp.float32),
            pltpu.VMEM_SHARED((N_NODES, D), jnp.float32),
            pltpu.VMEM((_SUP, _CHUNK), jnp.int32),
            pltpu.VMEM((_SUPE, D), jnp.float32),
            pltpu.VMEM((_CHUNK, D), jnp.float32),
            pltpu.VMEM((_ROWS_PER_SUB, D), jnp.float32),
            pltpu.SemaphoreType.DMA,
            pltpu.SemaphoreType.DMA,
        ],
        compiler_params=pltpu.CompilerParams(use_tc_tiling_on_sc=False),
    )(msg, dst2, zs, ones)


_MV_BLK = 2000


def _matvec_body(x_ref, a_ref, r_ref, out_ref):
    xb = x_ref[...].astype(jnp.bfloat16)
    xrep = jnp.dot(xb, r_ref[...], preferred_element_type=jnp.float32)
    prod = xrep * a_ref[...]
    t = prod[:, 0:128]
    for g in range(1, 8):
        t = t + prod[:, g * 128:(g + 1) * 128]
    out_ref[...] = (t[:, 0:32] + t[:, 32:64] + t[:, 64:96] + t[:, 96:128])


def _matvec(x_i, a2, r_mat):
    grid = N_EDGES // _MV_BLK
    return pl.pallas_call(
        _matvec_body,
        grid=(grid,),
        in_specs=[
            pl.BlockSpec((_MV_BLK, D), lambda i: (i, 0)),
            pl.BlockSpec((_MV_BLK, DD), lambda i: (i, 0)),
            pl.BlockSpec((D, DD), lambda i: (0, 0)),
        ],
        out_specs=pl.BlockSpec((_MV_BLK, D), lambda i: (i, 0)),
        out_shape=jax.ShapeDtypeStruct((N_EDGES, D), jnp.float32),
        compiler_params=pltpu.CompilerParams(
            dimension_semantics=("arbitrary",),
        ),
    )(x_i, a2, r_mat)


def _make_r() -> jax.Array:
    k = jnp.arange(D)[:, None]
    c = jnp.arange(DD)[None, :]
    return (c // D == k).astype(jnp.bfloat16)


_FIN_BLK = 2000


def _finalize_body(s_ref, c_ref, b_ref, out_ref):
    s = s_ref[0] + s_ref[1]
    c = c_ref[0] + c_ref[1]
    mean = s / jnp.maximum(c, 1.0)
    out_ref[...] = jnp.maximum(mean + b_ref[...], 0.0)


def _finalize(sums, counts, bias):
    grid = N_NODES // _FIN_BLK
    return pl.pallas_call(
        _finalize_body,
        grid=(grid,),
        in_specs=[
            pl.BlockSpec((2, _FIN_BLK, D), lambda i: (0, i, 0)),
            pl.BlockSpec((2, _FIN_BLK, D), lambda i: (0, i, 0)),
            pl.BlockSpec((1, D), lambda i: (0, 0)),
        ],
        out_specs=pl.BlockSpec((_FIN_BLK, D), lambda i: (i, 0)),
        out_shape=jax.ShapeDtypeStruct((N_NODES, D), jnp.float32),
    )(sums, counts, bias.reshape(1, D))


def kernel(node_states, edge_index, edge, a_in, bias):
    del edge
    src = edge_index[:, 0]
    dst = edge_index[:, 1]
    a2 = a_in.reshape(N_EDGES, DD)

    x_i = _sc_gather(node_states, src.reshape(_NSUP, _SUP, _CHUNK))

    msg = _matvec(x_i, a2, _make_r())

    sums2, counts2 = _sc_scatter(msg, dst.reshape(_NSUP, _SUP, _CHUNK))
    return _finalize(sums2, counts2, bias)


# speedup vs baseline: 2.6625x; 1.0120x over previous
import functools

import jax
import jax.numpy as jnp
from jax import lax
from jax.experimental import pallas as pl
from jax.experimental.pallas import tpu as pltpu
from jax.experimental.pallas import tpu_sc as plsc

N_NODES = 10000
N_EDGES = 160000
D = 32
DD = D * D

_NC = 2
_NS = 16
_NW = _NC * _NS
_CHUNK = 128
_NCHUNKS = N_EDGES // _CHUNK
_SUP = 10
_NSUP = _NCHUNKS // _SUP
_SUPE = _SUP * _CHUNK
_SC_MESH = dict(core_axis_name="c", subcore_axis_name="s",
                num_cores=_NC, num_subcores=_NS)


def _gather_body(ns_hbm, src_hbm, out_hbm, idx_v, rows_v, sem):
    wid = lax.axis_index("c") * _NS + lax.axis_index("s")

    def step(t, _):
        sup = wid + t * _NW

        @pl.when(sup < _NSUP)
        def _():
            pltpu.sync_copy(src_hbm.at[sup], idx_v)
            for j in range(_SUP):
                pltpu.async_copy(
                    ns_hbm.at[idx_v.at[j]],
                    rows_v.at[pl.ds(j * _CHUNK, _CHUNK)], sem)
            for j in range(_SUP):
                pltpu.make_async_copy(
                    ns_hbm.at[idx_v.at[j]],
                    rows_v.at[pl.ds(j * _CHUNK, _CHUNK)], sem).wait()
            pltpu.sync_copy(rows_v, out_hbm.at[pl.ds(sup * _SUPE, _SUPE)])
        return _

    lax.fori_loop(0, (_NSUP + _NW - 1) // _NW, step, None)


def _sc_gather(node_states, src3):
    mesh = plsc.VectorSubcoreMesh(**_SC_MESH)
    return pl.kernel(
        _gather_body,
        out_type=jax.ShapeDtypeStruct((N_EDGES, D), jnp.float32),
        mesh=mesh,
        scratch_types=[
            pltpu.VMEM((_SUP, _CHUNK), jnp.int32),
            pltpu.VMEM((_SUPE, D), jnp.float32),
            pltpu.SemaphoreType.DMA,
        ],
        compiler_params=pltpu.CompilerParams(use_tc_tiling_on_sc=False),
    )(node_states, src3)


_ROWS_PER_SUB = N_NODES // _NS


def _scatter_body(msg_hbm, dst_hbm, zs_hbm, ones_hbm, sums_hbm, cnts_hbm,
                  acc_sh, cnt_sh, idx_v, msg_v, ones_v, row_v, sem, sem2):
    cid = lax.axis_index("c")
    sid = lax.axis_index("s")
    wid = cid * _NS + sid
    rows0 = sid * _ROWS_PER_SUB

    pltpu.sync_copy(zs_hbm.at[pl.ds(rows0, _ROWS_PER_SUB)], row_v)
    pltpu.sync_copy(row_v, acc_sh.at[pl.ds(rows0, _ROWS_PER_SUB)])
    pltpu.sync_copy(row_v, cnt_sh.at[pl.ds(rows0, _ROWS_PER_SUB)])
    pltpu.sync_copy(ones_hbm, ones_v)
    plsc.subcore_barrier()

    def step(t, _):
        sup = wid + t * _NW

        @pl.when(sup < _NSUP)
        def _():
            pltpu.sync_copy(dst_hbm.at[sup], idx_v)
            pltpu.sync_copy(msg_hbm.at[pl.ds(sup * _SUPE, _SUPE)], msg_v)
            for j in range(_SUP):
                pltpu.async_copy(
                    msg_v.at[pl.ds(j * _CHUNK, _CHUNK)],
                    acc_sh.at[idx_v.at[j]], sem, add=True)
                pltpu.async_copy(
                    ones_v, cnt_sh.at[idx_v.at[j]], sem2, add=True)
            for j in range(_SUP):
                pltpu.make_async_copy(
                    msg_v.at[pl.ds(j * _CHUNK, _CHUNK)],
                    acc_sh.at[idx_v.at[j]], sem).wait()
                pltpu.make_async_copy(
                    ones_v, cnt_sh.at[idx_v.at[j]], sem2).wait()
        return _

    lax.fori_loop(0, (_NSUP + _NW - 1) // _NW, step, None)
    plsc.subcore_barrier()

    pltpu.sync_copy(acc_sh.at[pl.ds(rows0, _ROWS_PER_SUB)], row_v)
    pltpu.sync_copy(row_v, sums_hbm.at[cid, pl.ds(rows0, _ROWS_PER_SUB)])
    pltpu.sync_copy(cnt_sh.at[pl.ds(rows0, _ROWS_PER_SUB)], row_v)
    pltpu.sync_copy(row_v, cnts_hbm.at[cid, pl.ds(rows0, _ROWS_PER_SUB)])


def _sc_scatter(msg, dst2):
    mesh = plsc.VectorSubcoreMesh(**_SC_MESH)
    zs = jnp.zeros((N_NODES, D), jnp.float32)
    ones = jnp.ones((_CHUNK, D), jnp.float32)
    return pl.kernel(
        _scatter_body,
        out_type=(
            jax.ShapeDtypeStruct((_NC, N_NODES, D), jnp.float32),
            jax.ShapeDtypeStruct((_NC, N_NODES, D), jnp.float32),
        ),
        mesh=mesh,
        scratch_types=[
            pltpu.VMEM_SHARED((N_NODES, D), jnp.float32),
            pltpu.VMEM_SHARED((N_NODES, D), jnp.float32),
            pltpu.VMEM((_SUP, _CHUNK), jnp.int32),
            pltpu.VMEM((_SUPE, D), jnp.float32),
            pltpu.VMEM((_CHUNK, D), jnp.float32),
            pltpu.VMEM((_ROWS_PER_SUB, D), jnp.float32),
            pltpu.SemaphoreType.DMA,
            pltpu.SemaphoreType.DMA,
        ],
        compiler_params=pltpu.CompilerParams(use_tc_tiling_on_sc=False),
    )(msg, dst2, zs, ones)


_MV_BLK = 2000


def _matvec_body(x_ref, a_ref, r_ref, out_ref):
    xb = x_ref[...].astype(jnp.bfloat16)
    xrep = jnp.dot(xb, r_ref[...], preferred_element_type=jnp.float32)
    prod = xrep * a_ref[...]
    t = prod[:, 0:128]
    for g in range(1, 8):
        t = t + prod[:, g * 128:(g + 1) * 128]
    out_ref[...] = (t[:, 0:32] + t[:, 32:64] + t[:, 64:96] + t[:, 96:128])


def _matvec(x_i, a2, r_mat):
    grid = N_EDGES // _MV_BLK
    return pl.pallas_call(
        _matvec_body,
        grid=(grid,),
        in_specs=[
            pl.BlockSpec((_MV_BLK, D), lambda i: (i, 0)),
            pl.BlockSpec((_MV_BLK, DD), lambda i: (i, 0)),
            pl.BlockSpec((D, DD), lambda i: (0, 0)),
        ],
        out_specs=pl.BlockSpec((_MV_BLK, D), lambda i: (i, 0)),
        out_shape=jax.ShapeDtypeStruct((N_EDGES, D), jnp.float32),
        compiler_params=pltpu.CompilerParams(
            dimension_semantics=("arbitrary",),
        ),
    )(x_i, a2, r_mat)


def _make_r() -> jax.Array:
    k = jnp.arange(D)[:, None]
    c = jnp.arange(DD)[None, :]
    return (c // D == k).astype(jnp.bfloat16)


_FIN_BLK = 2000


def _finalize_body(s_ref, c_ref, b_ref, out_ref):
    s = s_ref[0] + s_ref[1]
    c = c_ref[0] + c_ref[1]
    mean = s / jnp.maximum(c, 1.0)
    out_ref[...] = jnp.maximum(mean + b_ref[...], 0.0)


def _finalize(sums, counts, bias):
    nq = N_NODES // 4
    return pl.pallas_call(
        _finalize_body,
        grid=(1,),
        in_specs=[
            pl.BlockSpec((2, nq, 128), lambda i: (0, 0, 0)),
            pl.BlockSpec((2, nq, 128), lambda i: (0, 0, 0)),
            pl.BlockSpec((1, 128), lambda i: (0, 0)),
        ],
        out_specs=pl.BlockSpec((nq, 128), lambda i: (0, 0)),
        out_shape=jax.ShapeDtypeStruct((nq, 128), jnp.float32),
    )(sums.reshape(2, nq, 128), counts.reshape(2, nq, 128),
      jnp.tile(bias, 4).reshape(1, 128))


def kernel(node_states, edge_index, edge, a_in, bias):
    del edge
    src = edge_index[:, 0]
    dst = edge_index[:, 1]
    a2 = a_in.reshape(N_EDGES, DD)

    x_i = _sc_gather(node_states, src.reshape(_NSUP, _SUP, _CHUNK))

    msg = _matvec(x_i, a2, _make_r())

    sums2, counts2 = _sc_scatter(msg, dst.reshape(_NSUP, _SUP, _CHUNK))
    return _finalize(sums2, counts2, bias).reshape(N_NODES, D)


# speedup vs baseline: 2.6634x; 1.0003x over previous
import jax
import jax.numpy as jnp
from jax import lax
from jax.experimental import pallas as pl
from jax.experimental.pallas import tpu as pltpu
from jax.experimental.pallas import tpu_sc as plsc

N_NODES = 10000
N_EDGES = 160000
D = 32
DD = D * D

_NC = 2
_NS = 16
_NW = _NC * _NS
_CHUNK = 128
_NCHUNKS = N_EDGES // _CHUNK
_SUP = 10
_NSUP = _NCHUNKS // _SUP
_SUPE = _SUP * _CHUNK
_SC_MESH = dict(core_axis_name="c", subcore_axis_name="s",
                num_cores=_NC, num_subcores=_NS)


def _gather_body(ns_hbm, src_hbm, out_hbm, idx_v, rows_v, sem):
    wid = lax.axis_index("c") * _NS + lax.axis_index("s")

    def step(t, _):
        sup = wid + t * _NW

        @pl.when(sup < _NSUP)
        def _():
            pltpu.sync_copy(src_hbm.at[sup], idx_v)
            for j in range(_SUP):
                pltpu.async_copy(
                    ns_hbm.at[idx_v.at[j]],
                    rows_v.at[pl.ds(j * _CHUNK, _CHUNK)], sem)
            for j in range(_SUP):
                pltpu.make_async_copy(
                    ns_hbm.at[idx_v.at[j]],
                    rows_v.at[pl.ds(j * _CHUNK, _CHUNK)], sem).wait()
            pltpu.sync_copy(rows_v, out_hbm.at[pl.ds(sup * _SUPE, _SUPE)])
        return _

    lax.fori_loop(0, (_NSUP + _NW - 1) // _NW, step, None)


def _sc_gather(node_states, src3):
    mesh = plsc.VectorSubcoreMesh(**_SC_MESH)
    return pl.kernel(
        _gather_body,
        out_type=jax.ShapeDtypeStruct((N_EDGES, D), jnp.float32),
        mesh=mesh,
        scratch_types=[
            pltpu.VMEM((_SUP, _CHUNK), jnp.int32),
            pltpu.VMEM((_SUPE, D), jnp.float32),
            pltpu.SemaphoreType.DMA,
        ],
        compiler_params=pltpu.CompilerParams(use_tc_tiling_on_sc=False),
    )(node_states, src3)


_ROWS_PER_SUB = N_NODES // _NS


def _scatter_body(msg_hbm, dst_hbm, zs_hbm, ones_hbm, sums_hbm, cnts_hbm,
                  acc_sh, cnt_sh, idx_v, msg_v, ones_v, row_v, sem, sem2):
    cid = lax.axis_index("c")
    sid = lax.axis_index("s")
    wid = cid * _NS + sid
    rows0 = sid * _ROWS_PER_SUB

    pltpu.sync_copy(zs_hbm.at[pl.ds(rows0, _ROWS_PER_SUB)], row_v)
    pltpu.sync_copy(row_v, acc_sh.at[pl.ds(rows0, _ROWS_PER_SUB)])
    pltpu.sync_copy(row_v, cnt_sh.at[pl.ds(rows0, _ROWS_PER_SUB)])
    pltpu.sync_copy(ones_hbm, ones_v)
    plsc.subcore_barrier()

    def step(t, _):
        sup = wid + t * _NW

        @pl.when(sup < _NSUP)
        def _():
            pltpu.sync_copy(dst_hbm.at[sup], idx_v)
            pltpu.sync_copy(msg_hbm.at[pl.ds(sup * _SUPE, _SUPE)], msg_v)
            for j in range(_SUP):
                pltpu.async_copy(
                    msg_v.at[pl.ds(j * _CHUNK, _CHUNK)],
                    acc_sh.at[idx_v.at[j]], sem, add=True)
                pltpu.async_copy(
                    ones_v, cnt_sh.at[idx_v.at[j]], sem2, add=True)
            for j in range(_SUP):
                pltpu.make_async_copy(
                    msg_v.at[pl.ds(j * _CHUNK, _CHUNK)],
                    acc_sh.at[idx_v.at[j]], sem).wait()
                pltpu.make_async_copy(
                    ones_v, cnt_sh.at[idx_v.at[j]], sem2).wait()
        return _

    lax.fori_loop(0, (_NSUP + _NW - 1) // _NW, step, None)
    plsc.subcore_barrier()

    pltpu.sync_copy(acc_sh.at[pl.ds(rows0, _ROWS_PER_SUB)], row_v)
    pltpu.sync_copy(row_v, sums_hbm.at[cid, pl.ds(rows0, _ROWS_PER_SUB)])
    pltpu.sync_copy(cnt_sh.at[pl.ds(rows0, _ROWS_PER_SUB)], row_v)
    pltpu.sync_copy(row_v, cnts_hbm.at[cid, pl.ds(rows0, _ROWS_PER_SUB)])


def _sc_scatter(msg, dst2):
    mesh = plsc.VectorSubcoreMesh(**_SC_MESH)
    zs = jnp.zeros((N_NODES, D), jnp.float32)
    ones = jnp.ones((_CHUNK, D), jnp.float32)
    return pl.kernel(
        _scatter_body,
        out_type=(
            jax.ShapeDtypeStruct((_NC, N_NODES, D), jnp.float32),
            jax.ShapeDtypeStruct((_NC, N_NODES, D), jnp.float32),
        ),
        mesh=mesh,
        scratch_types=[
            pltpu.VMEM_SHARED((N_NODES, D), jnp.float32),
            pltpu.VMEM_SHARED((N_NODES, D), jnp.float32),
            pltpu.VMEM((_SUP, _CHUNK), jnp.int32),
            pltpu.VMEM((_SUPE, D), jnp.float32),
            pltpu.VMEM((_CHUNK, D), jnp.float32),
            pltpu.VMEM((_ROWS_PER_SUB, D), jnp.float32),
            pltpu.SemaphoreType.DMA,
            pltpu.SemaphoreType.DMA,
        ],
        compiler_params=pltpu.CompilerParams(use_tc_tiling_on_sc=False),
    )(msg, dst2, zs, ones)


_MV_BLK = 2000


def _matvec_body(x_ref, a_ref, r_ref, out_ref):
    xb = x_ref[...].astype(jnp.bfloat16)
    xrep = jnp.dot(xb, r_ref[...], preferred_element_type=jnp.float32)
    prod = xrep * a_ref[...]
    t = prod[:, 0:128]
    for g in range(1, 8):
        t = t + prod[:, g * 128:(g + 1) * 128]
    out_ref[...] = (t[:, 0:32] + t[:, 32:64] + t[:, 64:96] + t[:, 96:128])


def _matvec(x_i, a2, r_mat):
    grid = N_EDGES // _MV_BLK
    return pl.pallas_call(
        _matvec_body,
        grid=(grid,),
        in_specs=[
            pl.BlockSpec((_MV_BLK, D), lambda i: (i, 0)),
            pl.BlockSpec((_MV_BLK, DD), lambda i: (i, 0)),
            pl.BlockSpec((D, DD), lambda i: (0, 0)),
        ],
        out_specs=pl.BlockSpec((_MV_BLK, D), lambda i: (i, 0)),
        out_shape=jax.ShapeDtypeStruct((N_EDGES, D), jnp.float32),
        compiler_params=pltpu.CompilerParams(
            dimension_semantics=("arbitrary",),
        ),
    )(x_i, a2, r_mat)


def _make_r() -> jax.Array:
    k = jnp.arange(D)[:, None]
    c = jnp.arange(DD)[None, :]
    return (c // D == k).astype(jnp.bfloat16)


_FIN_BLK = 2000


def _finalize_body(s_ref, c_ref, b_ref, out_ref):
    s = s_ref[0] + s_ref[1]
    c = c_ref[0] + c_ref[1]
    mean = s / jnp.maximum(c, 1.0)
    out_ref[...] = jnp.maximum(mean + b_ref[...], 0.0)


def _finalize(sums, counts, bias):
    nq = N_NODES // 4
    return pl.pallas_call(
        _finalize_body,
        grid=(1,),
        in_specs=[
            pl.BlockSpec((2, nq, 128), lambda i: (0, 0, 0)),
            pl.BlockSpec((2, nq, 128), lambda i: (0, 0, 0)),
            pl.BlockSpec((1, 128), lambda i: (0, 0)),
        ],
        out_specs=pl.BlockSpec((nq, 128), lambda i: (0, 0)),
        out_shape=jax.ShapeDtypeStruct((nq, 128), jnp.float32),
    )(sums.reshape(2, nq, 128), counts.reshape(2, nq, 128),
      jnp.tile(bias, 4).reshape(1, 128))


def kernel(node_states, edge_index, edge, a_in, bias):
    del edge
    src = edge_index[:, 0]
    dst = edge_index[:, 1]
    a2 = a_in.reshape(N_EDGES, DD)

    x_i = _sc_gather(node_states, src.reshape(_NSUP, _SUP, _CHUNK))

    msg = _matvec(x_i, a2, _make_r())

    sums2, counts2 = _sc_scatter(msg, dst.reshape(_NSUP, _SUP, _CHUNK))
    return _finalize(sums2, counts2, bias).reshape(N_NODES, D)
